# baseline (speedup 1.0000x reference)
import functools
import numpy as np
import jax
from jax import lax
import jax.numpy as jnp
from jax.experimental import pallas as pl
from jax.experimental.pallas import tpu as pltpu


def _body(C, E, NBK, x_ref, wg_ref, w1_ref, b1_ref, w2_ref, b2_ref,
          out_ref, slot_s, w_s, s_acc, dx_s):
    i = pl.program_id(0)
    e = i // 2
    half = i % 2
    N = x_ref.shape[0]

    @pl.when(i == 0)
    def _():
        xf = x_ref[...]
        logits = jnp.dot(xf, wg_ref[...], preferred_element_type=jnp.float32)
        m = jnp.max(logits, axis=1, keepdims=True)
        gv = 1.0 / jnp.sum(jnp.exp(logits - m), axis=1, keepdims=True)
        e_iota = lax.broadcasted_iota(jnp.int32, logits.shape, 1)
        idx = jnp.min(jnp.where(logits == m, e_iota, E), axis=1, keepdims=True)
        oh = (e_iota == idx).astype(jnp.float32)
        BS = N // NBK
        r = lax.broadcasted_iota(jnp.int32, (BS, BS), 0)
        c2 = lax.broadcasted_iota(jnp.int32, (BS, BS), 1)
        tri = (c2 < r).astype(jnp.float32)
        base = jnp.zeros((1, E), jnp.float32)
        for b in range(NBK):
            sl = slice(b * BS, (b + 1) * BS)
            ohb = oh[sl, :]
            cum = jnp.dot(tri, ohb, preferred_element_type=jnp.float32) + base
            posb = jnp.sum(cum * ohb, axis=1, keepdims=True).astype(jnp.int32)
            keepb = posb < C
            slot_s[sl, :] = jnp.where(keepb, idx[sl, :] * C + posb, E * C)
            w_s[sl, :] = jnp.where(keepb, gv[sl, :], 0.0)
            base = base + jnp.sum(ohb, axis=0, keepdims=True)

    slot_col = slot_s[...]
    c_iota = lax.broadcasted_iota(jnp.int32, (N, C), 1)
    P = (slot_col == e * C + c_iota).astype(jnp.float32)

    @pl.when(half == 0)
    def _():
        dx_s[...] = lax.dot_general(
            P, x_ref[...], (((0,), (0,)), ((), ())),
            preferred_element_type=jnp.float32)

    h = jnp.maximum(
        jnp.dot(dx_s[...], w1_ref[0], preferred_element_type=jnp.float32)
        + b1_ref[0], 0.0)
    w2s = jnp.sum(w2_ref[0], axis=1, keepdims=True)
    val = jnp.dot(h, w2s, preferred_element_type=jnp.float32)
    val = val + jnp.sum(b2_ref[0]) * 0.5
    contrib = jnp.dot(P, val, preferred_element_type=jnp.float32) \
        * w_s[...]

    @pl.when(i == 0)
    def _():
        s_acc[...] = contrib

    @pl.when(i > 0)
    def _():
        s_acc[...] = s_acc[...] + contrib

    @pl.when(i == 2 * E - 1)
    def _():
        s = s_acc[...]
        mx = jnp.max(s, axis=0, keepdims=True)
        lse = jnp.log(jnp.sum(jnp.exp(s - mx), axis=0, keepdims=True)) + mx
        out_ref[...] = s - lse


def kernel(x, Wg, W1, b1, W2, b2):
    B_, T_, D_ = x.shape
    N = B_ * T_
    E_ = Wg.shape[1]
    H_ = W1.shape[2]
    HH = H_ // 2
    C = int(np.ceil(N * 1.25 / E_))
    xf = x.reshape(N, D_)

    out = pl.pallas_call(
        functools.partial(_body, C, E_, 16),
        grid=(2 * E_,),
        in_specs=[
            pl.BlockSpec((N, D_), lambda i: (0, 0)),
            pl.BlockSpec((D_, E_), lambda i: (0, 0)),
            pl.BlockSpec((1, D_, HH), lambda i: (i // 2, 0, i % 2)),
            pl.BlockSpec((1, 1, HH), lambda i: (i // 2, 0, i % 2)),
            pl.BlockSpec((1, HH, D_), lambda i: (i // 2, i % 2, 0)),
            pl.BlockSpec((1, 1, D_), lambda i: (i // 2, 0, 0)),
        ],
        out_specs=pl.BlockSpec((N, 1), lambda i: (0, 0)),
        out_shape=jax.ShapeDtypeStruct((N, 1), jnp.float32),
        scratch_shapes=[pltpu.VMEM((N, 1), jnp.int32),
                        pltpu.VMEM((N, 1), jnp.float32),
                        pltpu.VMEM((N, 1), jnp.float32),
                        pltpu.VMEM((C, D_), jnp.float32)],
    )(xf, Wg, W1, b1.reshape(E_, 1, H_), W2, b2.reshape(E_, 1, D_))
    return out.reshape(B_, T_)


# speedup vs baseline: 1.3261x; 1.3261x over previous
import functools
import numpy as np
import jax
from jax import lax
import jax.numpy as jnp
from jax.experimental import pallas as pl
from jax.experimental.pallas import tpu as pltpu


def _body(C, E, NBK, x_ref, wg_ref, w1_ref, b1_ref, w2_ref, b2_ref,
          out_ref, slot_s, w_s, s_acc):
    e = pl.program_id(0)
    N = x_ref.shape[0]

    @pl.when(e == 0)
    def _():
        xf = x_ref[...]
        logits = jnp.dot(xf, wg_ref[...], preferred_element_type=jnp.float32)
        m = jnp.max(logits, axis=1, keepdims=True)
        gv = 1.0 / jnp.sum(jnp.exp(logits - m), axis=1, keepdims=True)
        e_iota = lax.broadcasted_iota(jnp.int32, logits.shape, 1)
        idx = jnp.min(jnp.where(logits == m, e_iota, E), axis=1, keepdims=True)
        oh = (e_iota == idx).astype(jnp.float32)
        BS = N // NBK
        r = lax.broadcasted_iota(jnp.int32, (BS, BS), 0)
        c2 = lax.broadcasted_iota(jnp.int32, (BS, BS), 1)
        tri = (c2 < r).astype(jnp.float32)
        base = jnp.zeros((1, E), jnp.float32)
        for b in range(NBK):
            sl = slice(b * BS, (b + 1) * BS)
            ohb = oh[sl, :]
            cum = jnp.dot(tri, ohb, preferred_element_type=jnp.float32) + base
            posb = jnp.sum(cum * ohb, axis=1, keepdims=True).astype(jnp.int32)
            keepb = posb < C
            slot_s[sl, :] = jnp.where(keepb, idx[sl, :] * C + posb, E * C)
            w_s[sl, :] = jnp.where(keepb, gv[sl, :], 0.0)
            base = base + jnp.sum(ohb, axis=0, keepdims=True)

    slot_col = slot_s[...]
    c_iota = lax.broadcasted_iota(jnp.int32, (N, C), 1)
    P = (slot_col == e * C + c_iota).astype(jnp.float32)
    dispx = lax.dot_general(
        P, x_ref[...], (((0,), (0,)), ((), ())),
        preferred_element_type=jnp.float32)
    h = jnp.maximum(
        jnp.dot(dispx, w1_ref[0], preferred_element_type=jnp.float32)
        + b1_ref[0], 0.0)
    w2s = jnp.sum(w2_ref[0], axis=1, keepdims=True)
    val = jnp.dot(h, w2s, preferred_element_type=jnp.float32) \
        + jnp.sum(b2_ref[0])
    contrib = jnp.dot(P, val, preferred_element_type=jnp.float32) \
        * w_s[...]

    @pl.when(e == 0)
    def _():
        s_acc[...] = contrib

    @pl.when(e > 0)
    def _():
        s_acc[...] = s_acc[...] + contrib

    @pl.when(e == E - 1)
    def _():
        s = s_acc[...]
        mx = jnp.max(s, axis=0, keepdims=True)
        lse = jnp.log(jnp.sum(jnp.exp(s - mx), axis=0, keepdims=True)) + mx
        out_ref[...] = s - lse


def kernel(x, Wg, W1, b1, W2, b2):
    B_, T_, D_ = x.shape
    N = B_ * T_
    E_ = Wg.shape[1]
    H_ = W1.shape[2]
    C = int(np.ceil(N * 1.25 / E_))
    xf = x.reshape(N, D_)

    out = pl.pallas_call(
        functools.partial(_body, C, E_, 16),
        grid=(E_,),
        in_specs=[
            pl.BlockSpec((N, D_), lambda e: (0, 0)),
            pl.BlockSpec((D_, E_), lambda e: (0, 0)),
            pl.BlockSpec((1, D_, H_), lambda e: (e, 0, 0)),
            pl.BlockSpec((1, 1, H_), lambda e: (e, 0, 0)),
            pl.BlockSpec((1, H_, D_), lambda e: (e, 0, 0)),
            pl.BlockSpec((1, 1, D_), lambda e: (e, 0, 0)),
        ],
        out_specs=pl.BlockSpec((N, 1), lambda e: (0, 0)),
        out_shape=jax.ShapeDtypeStruct((N, 1), jnp.float32),
        scratch_shapes=[pltpu.VMEM((N, 1), jnp.int32),
                        pltpu.VMEM((N, 1), jnp.float32),
                        pltpu.VMEM((N, 1), jnp.float32)],
    )(xf, Wg, W1, b1.reshape(E_, 1, H_), W2, b2.reshape(E_, 1, D_))
    return out.reshape(B_, T_)
